# baseline (speedup 1.0000x reference)
import jax
import jax.numpy as jnp
from jax import lax
from jax.experimental import pallas as pl
from jax.experimental.pallas import tpu as pltpu
from jax.experimental.pallas import tpu_sc as plsc

_N = 10000
_E = 320000
_NC = 2
_NS = 16
_NW = _NC * _NS
_NP = 10240
_RPT = _NP // _NS
_DH = 64
_NB = 8
_DEPTH = 4
_RB = 2000

_CHW = 96
_EPW = _NS * _CHW * 209
_KJS = _EPW // _NS // _CHW

_CHN = 128
_EPN = _NW * _CHN * 80
_KJW = _EPN // _NW // _CHN

_mesh = plsc.VectorSubcoreMesh(core_axis_name="c", subcore_axis_name="s")
_sc_params = pltpu.CompilerParams(use_tc_tiling_on_sc=False)


def _ring_pipeline(tbl, row_v, col_v, acc_sh, msgs, gs, ss, kj, ch):
  def idx(ref, jj):
    return ref.at[pl.ds(jj * ch, ch)]

  def start_gather(jj, b):
    pltpu.async_copy(tbl.at[idx(row_v, jj)], msgs[b], gs[b])

  def wait_gather(jj, b):
    pltpu.make_async_copy(tbl.at[idx(row_v, jj)], msgs[b], gs[b]).wait()

  def start_scatter(jj, b):
    pltpu.async_copy(msgs[b], acc_sh.at[idx(col_v, jj)], ss[b], add=True)

  def drain_scatter(b):
    pltpu.make_async_copy(msgs[b], acc_sh.at[idx(col_v, 0)], ss[b]).wait()

  def step(jj, b, need_drain, need_start):
    wait_gather(jj, b)
    start_scatter(jj, b)
    if need_start:
      bd = (b + _DEPTH) % _NB
      if need_drain:
        drain_scatter(bd)
      start_gather(jj + _DEPTH, bd)

  for t in range(_DEPTH):
    start_gather(t, t)
  for jj in range(_NB):
    step(jj, jj % _NB, need_drain=(jj >= _NB - _DEPTH),
         need_start=(jj + _DEPTH < kj))

  main_hi = _NB + 8 * ((kj - _NB - _DEPTH) // 8)

  @pl.loop(_NB, main_hi, step=8)
  def _(jj0):
    for u in range(8):
      step(jj0 + u, u, need_drain=True, need_start=True)

  for jj in range(main_hi, kj):
    step(jj, jj % _NB, need_drain=True, need_start=(jj + _DEPTH < kj))
  for b in range(_NB):
    drain_scatter(b)


def _sc_scratch(kj, ch, d):
  return ([pltpu.VMEM((kj * ch,), jnp.int32),
           pltpu.VMEM((kj * ch,), jnp.int32)]
          + [pltpu.VMEM((ch, d), jnp.float32) for _ in range(_NB)]
          + [pltpu.SemaphoreType.DMA for _ in range(2 * _NB)])


def _seg_sum_split(hpp2, rowx2, col_f, zeros):
  @pl.kernel(
      out_type=jax.ShapeDtypeStruct((_NP, 2 * _DH), jnp.float32),
      mesh=_mesh,
      compiler_params=_sc_params,
      scratch_types=_sc_scratch(_KJS, _CHW, _DH)
      + [pltpu.VMEM_SHARED((_NP, _DH), jnp.float32)],
  )
  def k(hpp_hbm, row_hbm, col_hbm, zeros_hbm, out_hbm,
        row_v, col_v, *rest):
    msgs, gs, ss, acc_sh = (rest[:_NB], rest[_NB:2 * _NB],
                            rest[2 * _NB:3 * _NB], rest[3 * _NB])
    cid = lax.axis_index("c")
    sid = lax.axis_index("s")
    epw = _KJS * _CHW
    pltpu.sync_copy(zeros_hbm, acc_sh.at[pl.ds(sid * _RPT, _RPT)])
    pltpu.sync_copy(row_hbm.at[pl.ds(cid * _EPW + sid * epw, epw)], row_v)
    pltpu.sync_copy(col_hbm.at[pl.ds(sid * epw, epw)], col_v)
    plsc.subcore_barrier()
    _ring_pipeline(hpp_hbm, row_v, col_v, acc_sh, msgs, gs, ss,
                   _KJS, _CHW)
    plsc.subcore_barrier()
    pltpu.sync_copy(acc_sh.at[pl.ds(sid * _RPT, _RPT)],
                    out_hbm.at[pl.ds(sid * _RPT, _RPT),
                               pl.ds(cid * _DH, _DH)])

  return k(hpp2, rowx2, col_f, zeros)


def _seg_sum_part(hpp, row_f, col_f, zeros, D):
  @pl.kernel(
      out_type=jax.ShapeDtypeStruct((_NC, _NP, D), jnp.float32),
      mesh=_mesh,
      compiler_params=_sc_params,
      scratch_types=_sc_scratch(_KJW, _CHN, D)
      + [pltpu.VMEM_SHARED((_NP, D), jnp.float32)],
  )
  def k(hpp_hbm, row_hbm, col_hbm, zeros_hbm, out_hbm,
        row_v, col_v, *rest):
    msgs, gs, ss, acc_sh = (rest[:_NB], rest[_NB:2 * _NB],
                            rest[2 * _NB:3 * _NB], rest[3 * _NB])
    cid = lax.axis_index("c")
    sid = lax.axis_index("s")
    wid = sid * _NC + cid
    epw = _KJW * _CHN
    pltpu.sync_copy(zeros_hbm, acc_sh.at[pl.ds(sid * _RPT, _RPT)])
    pltpu.sync_copy(row_hbm.at[pl.ds(wid * epw, epw)], row_v)
    pltpu.sync_copy(col_hbm.at[pl.ds(wid * epw, epw)], col_v)
    plsc.subcore_barrier()
    _ring_pipeline(hpp_hbm, row_v, col_v, acc_sh, msgs, gs, ss,
                   _KJW, _CHN)
    plsc.subcore_barrier()
    pltpu.sync_copy(acc_sh.at[pl.ds(sid * _RPT, _RPT)],
                    out_hbm.at[cid, pl.ds(sid * _RPT, _RPT)])

  return k(hpp, row_f, col_f, zeros)


def _deg_hist(col_f, ones, zeros):
  @pl.kernel(
      out_type=jax.ShapeDtypeStruct((_NP, 2 * _DH), jnp.float32),
      mesh=_mesh,
      compiler_params=_sc_params,
      scratch_types=[
          pltpu.VMEM((_KJW * _CHN,), jnp.int32),
          pltpu.VMEM((_CHN, 16), jnp.float32),
          pltpu.SemaphoreType.DMA,
          pltpu.VMEM_SHARED((_NP, 16), jnp.float32),
      ],
  )
  def k(col_hbm, ones_hbm, zeros_hbm, out_hbm, col_v, ones_v, sem, acc_sh):
    cid = lax.axis_index("c")
    sid = lax.axis_index("s")
    wid = sid * _NC + cid
    epw = _KJW * _CHN
    pltpu.sync_copy(zeros_hbm, acc_sh.at[pl.ds(sid * _RPT, _RPT)])
    pltpu.sync_copy(col_hbm.at[pl.ds(wid * epw, epw)], col_v)
    pltpu.sync_copy(ones_hbm, ones_v)
    plsc.subcore_barrier()

    @pl.loop(0, _KJW)
    def _(j):
      pltpu.async_copy(ones_v, acc_sh.at[col_v.at[pl.ds(j * _CHN, _CHN)]],
                       sem, add=True)

    @pl.loop(0, _KJW)
    def _(j):
      pltpu.make_async_copy(ones_v, acc_sh.at[col_v.at[pl.ds(0, _CHN)]],
                            sem).wait()

    plsc.subcore_barrier()
    pltpu.sync_copy(acc_sh.at[pl.ds(sid * _RPT, _RPT)],
                    out_hbm.at[pl.ds(sid * _RPT, _RPT),
                               pl.ds(16 * cid, 16)])

  return k(col_f, ones, zeros)


def _mm_plain(x, W):
  def body(x_ref, w_ref, o_ref):
    o_ref[...] = lax.dot_general(x_ref[...], w_ref[...],
                                 (((1,), (0,)), ((), ())),
                                 preferred_element_type=jnp.float32,
                                 precision=lax.Precision.HIGHEST)

  K, Dout = W.shape
  return pl.pallas_call(
      body,
      grid=(_N // _RB,),
      in_specs=[pl.BlockSpec((_RB, K), lambda i: (i, 0)),
                pl.BlockSpec((K, Dout), lambda i: (0, 0))],
      out_specs=pl.BlockSpec((_RB, Dout), lambda i: (i, 0)),
      out_shape=jax.ShapeDtypeStruct((_N, Dout), jnp.float32),
  )(x, W)


def _g_scale(dacc, h0):
  def body(d_ref, h_ref, g_ref, o_ref):
    deg = d_ref[:, :1] + d_ref[:, 16:17] + 1.0
    g = jnp.broadcast_to(lax.rsqrt(deg), (_RB, 2 * _DH))
    g_ref[...] = g
    o_ref[...] = h_ref[...] * g

  return pl.pallas_call(
      body,
      grid=(_N // _RB,),
      in_specs=[pl.BlockSpec((_RB, 2 * _DH), lambda i: (i, 0)),
                pl.BlockSpec((_RB, 2 * _DH), lambda i: (i, 0))],
      out_specs=[pl.BlockSpec((_RB, 2 * _DH), lambda i: (i, 0)),
                 pl.BlockSpec((_RB, 2 * _DH), lambda i: (i, 0))],
      out_shape=[jax.ShapeDtypeStruct((_N, 2 * _DH), jnp.float32),
                 jax.ShapeDtypeStruct((_N, 2 * _DH), jnp.float32)],
  )(dacc, h0)


def _mid(acc, hpp, g, b, W):
  def body(a_ref, h_ref, g_ref, b_ref, w_ref, o_ref):
    g = g_ref[...]
    t = jnp.maximum(g * (a_ref[...] + h_ref[...]) + b_ref[...], 0.0)
    h = lax.dot_general(t, w_ref[...], (((1,), (0,)), ((), ())),
                        preferred_element_type=jnp.float32,
                        precision=lax.Precision.HIGHEST)
    o_ref[...] = h * g[:, :h.shape[1]]

  K, Dout = W.shape
  return pl.pallas_call(
      body,
      grid=(_N // _RB,),
      in_specs=[pl.BlockSpec((_RB, K), lambda i: (i, 0)),
                pl.BlockSpec((_RB, K), lambda i: (i, 0)),
                pl.BlockSpec((_RB, 2 * _DH), lambda i: (i, 0)),
                pl.BlockSpec((1, K), lambda i: (0, 0)),
                pl.BlockSpec((K, Dout), lambda i: (0, 0))],
      out_specs=pl.BlockSpec((_RB, Dout), lambda i: (i, 0)),
      out_shape=jax.ShapeDtypeStruct((_N, Dout), jnp.float32),
  )(acc, hpp, g, b, W)


def _final(acc, hpp, g, b):
  def body(a_ref, h_ref, g_ref, b_ref, o_ref):
    g = g_ref[...][:, :h_ref.shape[1]]
    o_ref[...] = g * (a_ref[0] + a_ref[1] + h_ref[...]) + b_ref[...]

  D = hpp.shape[1]
  return pl.pallas_call(
      body,
      grid=(_N // _RB,),
      in_specs=[pl.BlockSpec((_NC, _RB, D), lambda i: (0, i, 0)),
                pl.BlockSpec((_RB, D), lambda i: (i, 0)),
                pl.BlockSpec((_RB, 2 * _DH), lambda i: (i, 0)),
                pl.BlockSpec((1, D), lambda i: (0, 0))],
      out_specs=pl.BlockSpec((_RB, D), lambda i: (i, 0)),
      out_shape=jax.ShapeDtypeStruct((_N, D), jnp.float32),
  )(acc, hpp, g, b)


def kernel(x, edge_index, W0, b0, W1, b1, W2, b2, W3, b3, W4, b4):
  pad_rw = jnp.zeros((_EPW - _E,), jnp.int32)
  pad_cw = jnp.full((_EPW - _E,), _NP - 1, jnp.int32)
  row_w = jnp.concatenate([edge_index[0], pad_rw])
  col_w = jnp.concatenate([edge_index[1], pad_cw])
  rowx2 = jnp.concatenate([2 * row_w, 2 * row_w + 1])
  pad_rn = jnp.zeros((_EPN - _E,), jnp.int32)
  pad_cn = jnp.full((_EPN - _E,), _NP - 1, jnp.int32)
  row_n = jnp.concatenate([edge_index[0], pad_rn])
  col_n = jnp.concatenate([edge_index[1], pad_cn])

  zeros64 = jnp.zeros((_RPT, _DH), jnp.float32)
  zeros16 = jnp.zeros((_RPT, 16), jnp.float32)
  ones16 = jnp.ones((_CHN, 16), jnp.float32)

  dacc = _deg_hist(col_n, ones16, zeros16)
  h0 = _mm_plain(x, W0)
  g, hpp = _g_scale(dacc, h0)

  W4p = jnp.pad(W4, ((0, 0), (0, 14)))
  b4p = jnp.pad(b4, (0, 14)).reshape(1, 16)

  bs = (b0.reshape(1, -1), b1.reshape(1, -1), b2.reshape(1, -1),
        b3.reshape(1, -1))
  Ws = (W1, W2, W3, W4p)
  for i in range(4):
    acc = _seg_sum_split(hpp.reshape(2 * _N, _DH), rowx2, col_w, zeros64)
    hpp = _mid(acc, hpp, g, bs[i], Ws[i])
  acc = _seg_sum_part(hpp, row_n, col_n, zeros16, 16)
  out16 = _final(acc, hpp, g, b4p)
  return out16[:, :2]


# speedup vs baseline: 1.2286x; 1.2286x over previous
import jax
import jax.numpy as jnp
from jax import lax
from jax.experimental import pallas as pl
from jax.experimental.pallas import tpu as pltpu
from jax.experimental.pallas import tpu_sc as plsc

_N = 10000
_E = 320000
_NC = 2
_NS = 16
_NW = _NC * _NS
_NP = 10240
_RPT = _NP // _NS
_DH = 64
_NB = 8
_DEPTH = 4
_RB = 2000

_CHW = 80
_EPW = _NS * _CHW * 250
_KJS = _EPW // _NS // _CHW

_CHN = 128
_EPN = _NW * _CHN * 80
_KJW = _EPN // _NW // _CHN

_mesh = plsc.VectorSubcoreMesh(core_axis_name="c", subcore_axis_name="s")
_sc_params = pltpu.CompilerParams(use_tc_tiling_on_sc=False)


def _ring_pipeline(tbl, row_v, col_v, acc_sh, msgs, gs, ss, kj, ch):
  def idx(ref, jj):
    return ref.at[pl.ds(jj * ch, ch)]

  def start_gather(jj, b):
    pltpu.async_copy(tbl.at[idx(row_v, jj)], msgs[b], gs[b])

  def wait_gather(jj, b):
    pltpu.make_async_copy(tbl.at[idx(row_v, jj)], msgs[b], gs[b]).wait()

  def start_scatter(jj, b):
    pltpu.async_copy(msgs[b], acc_sh.at[idx(col_v, jj)], ss[b], add=True)

  def drain_scatter(b):
    pltpu.make_async_copy(msgs[b], acc_sh.at[idx(col_v, 0)], ss[b]).wait()

  def step(jj, b, need_drain, need_start):
    wait_gather(jj, b)
    start_scatter(jj, b)
    if need_start:
      bd = (b + _DEPTH) % _NB
      if need_drain:
        drain_scatter(bd)
      start_gather(jj + _DEPTH, bd)

  for t in range(_DEPTH):
    start_gather(t, t)
  for jj in range(_NB):
    step(jj, jj % _NB, need_drain=(jj >= _NB - _DEPTH),
         need_start=(jj + _DEPTH < kj))

  main_hi = _NB + 8 * ((kj - _NB - _DEPTH) // 8)

  @pl.loop(_NB, main_hi, step=8)
  def _(jj0):
    for u in range(8):
      step(jj0 + u, u, need_drain=True, need_start=True)

  for jj in range(main_hi, kj):
    step(jj, jj % _NB, need_drain=True, need_start=(jj + _DEPTH < kj))
  for b in range(_NB):
    drain_scatter(b)


def _sc_scratch(kj, ch, d):
  return ([pltpu.VMEM((kj * ch,), jnp.int32),
           pltpu.VMEM((kj * ch,), jnp.int32)]
          + [pltpu.VMEM((ch, d), jnp.float32) for _ in range(_NB)]
          + [pltpu.SemaphoreType.DMA for _ in range(2 * _NB)])


def _seg_sum_split(hpp2, rowx2, col_f, zeros):
  @pl.kernel(
      out_type=jax.ShapeDtypeStruct((_NP, 2 * _DH), jnp.float32),
      mesh=_mesh,
      compiler_params=_sc_params,
      scratch_types=_sc_scratch(_KJS, _CHW, _DH)
      + [pltpu.VMEM_SHARED((_NP, _DH), jnp.float32)],
  )
  def k(hpp_hbm, row_hbm, col_hbm, zeros_hbm, out_hbm,
        row_v, col_v, *rest):
    msgs, gs, ss, acc_sh = (rest[:_NB], rest[_NB:2 * _NB],
                            rest[2 * _NB:3 * _NB], rest[3 * _NB])
    cid = lax.axis_index("c")
    sid = lax.axis_index("s")
    epw = _KJS * _CHW
    pltpu.sync_copy(zeros_hbm, acc_sh.at[pl.ds(sid * _RPT, _RPT)])
    pltpu.sync_copy(row_hbm.at[pl.ds(cid * _EPW + sid * epw, epw)], row_v)
    pltpu.sync_copy(col_hbm.at[pl.ds(sid * epw, epw)], col_v)
    plsc.subcore_barrier()
    _ring_pipeline(hpp_hbm, row_v, col_v, acc_sh, msgs, gs, ss,
                   _KJS, _CHW)
    plsc.subcore_barrier()
    pltpu.sync_copy(acc_sh.at[pl.ds(sid * _RPT, _RPT)],
                    out_hbm.at[pl.ds(sid * _RPT, _RPT),
                               pl.ds(cid * _DH, _DH)])

  return k(hpp2, rowx2, col_f, zeros)


def _seg_sum_part(hpp, row_f, col_f, zeros, D):
  @pl.kernel(
      out_type=jax.ShapeDtypeStruct((_NC, _NP, D), jnp.float32),
      mesh=_mesh,
      compiler_params=_sc_params,
      scratch_types=_sc_scratch(_KJW, _CHN, D)
      + [pltpu.VMEM_SHARED((_NP, D), jnp.float32)],
  )
  def k(hpp_hbm, row_hbm, col_hbm, zeros_hbm, out_hbm,
        row_v, col_v, *rest):
    msgs, gs, ss, acc_sh = (rest[:_NB], rest[_NB:2 * _NB],
                            rest[2 * _NB:3 * _NB], rest[3 * _NB])
    cid = lax.axis_index("c")
    sid = lax.axis_index("s")
    wid = sid * _NC + cid
    epw = _KJW * _CHN
    pltpu.sync_copy(zeros_hbm, acc_sh.at[pl.ds(sid * _RPT, _RPT)])
    pltpu.sync_copy(row_hbm.at[pl.ds(wid * epw, epw)], row_v)
    pltpu.sync_copy(col_hbm.at[pl.ds(wid * epw, epw)], col_v)
    plsc.subcore_barrier()
    _ring_pipeline(hpp_hbm, row_v, col_v, acc_sh, msgs, gs, ss,
                   _KJW, _CHN)
    plsc.subcore_barrier()
    pltpu.sync_copy(acc_sh.at[pl.ds(sid * _RPT, _RPT)],
                    out_hbm.at[cid, pl.ds(sid * _RPT, _RPT)])

  return k(hpp, row_f, col_f, zeros)


def _deg_hist(col_f, ones, zeros):
  @pl.kernel(
      out_type=jax.ShapeDtypeStruct((_NP, 2 * _DH), jnp.float32),
      mesh=_mesh,
      compiler_params=_sc_params,
      scratch_types=[
          pltpu.VMEM((_KJW * _CHN,), jnp.int32),
          pltpu.VMEM((_CHN, 16), jnp.float32),
          pltpu.SemaphoreType.DMA,
          pltpu.VMEM_SHARED((_NP, 16), jnp.float32),
      ],
  )
  def k(col_hbm, ones_hbm, zeros_hbm, out_hbm, col_v, ones_v, sem, acc_sh):
    cid = lax.axis_index("c")
    sid = lax.axis_index("s")
    wid = sid * _NC + cid
    epw = _KJW * _CHN
    pltpu.sync_copy(zeros_hbm, acc_sh.at[pl.ds(sid * _RPT, _RPT)])
    pltpu.sync_copy(col_hbm.at[pl.ds(wid * epw, epw)], col_v)
    pltpu.sync_copy(ones_hbm, ones_v)
    plsc.subcore_barrier()

    @pl.loop(0, _KJW)
    def _(j):
      pltpu.async_copy(ones_v, acc_sh.at[col_v.at[pl.ds(j * _CHN, _CHN)]],
                       sem, add=True)

    @pl.loop(0, _KJW)
    def _(j):
      pltpu.make_async_copy(ones_v, acc_sh.at[col_v.at[pl.ds(0, _CHN)]],
                            sem).wait()

    plsc.subcore_barrier()
    pltpu.sync_copy(acc_sh.at[pl.ds(sid * _RPT, _RPT)],
                    out_hbm.at[pl.ds(sid * _RPT, _RPT),
                               pl.ds(16 * cid, 16)])

  return k(col_f, ones, zeros)


def _mm_plain(x, W):
  def body(x_ref, w_ref, o_ref):
    o_ref[...] = lax.dot_general(x_ref[...], w_ref[...],
                                 (((1,), (0,)), ((), ())),
                                 preferred_element_type=jnp.float32,
                                 precision=lax.Precision.HIGHEST)

  K, Dout = W.shape
  return pl.pallas_call(
      body,
      grid=(_N // _RB,),
      in_specs=[pl.BlockSpec((_RB, K), lambda i: (i, 0)),
                pl.BlockSpec((K, Dout), lambda i: (0, 0))],
      out_specs=pl.BlockSpec((_RB, Dout), lambda i: (i, 0)),
      out_shape=jax.ShapeDtypeStruct((_N, Dout), jnp.float32),
  )(x, W)


def _g_scale(dacc, h0):
  def body(d_ref, h_ref, g_ref, o_ref):
    deg = d_ref[:, :1] + d_ref[:, 16:17] + 1.0
    g = jnp.broadcast_to(lax.rsqrt(deg), (_RB, 2 * _DH))
    g_ref[...] = g
    o_ref[...] = h_ref[...] * g

  return pl.pallas_call(
      body,
      grid=(_N // _RB,),
      in_specs=[pl.BlockSpec((_RB, 2 * _DH), lambda i: (i, 0)),
                pl.BlockSpec((_RB, 2 * _DH), lambda i: (i, 0))],
      out_specs=[pl.BlockSpec((_RB, 2 * _DH), lambda i: (i, 0)),
                 pl.BlockSpec((_RB, 2 * _DH), lambda i: (i, 0))],
      out_shape=[jax.ShapeDtypeStruct((_N, 2 * _DH), jnp.float32),
                 jax.ShapeDtypeStruct((_N, 2 * _DH), jnp.float32)],
  )(dacc, h0)


def _mid(acc, hpp, g, b, W):
  def body(a_ref, h_ref, g_ref, b_ref, w_ref, o_ref):
    g = g_ref[...]
    t = jnp.maximum(g * (a_ref[...] + h_ref[...]) + b_ref[...], 0.0)
    h = lax.dot_general(t, w_ref[...], (((1,), (0,)), ((), ())),
                        preferred_element_type=jnp.float32,
                        precision=lax.Precision.HIGHEST)
    o_ref[...] = h * g[:, :h.shape[1]]

  K, Dout = W.shape
  return pl.pallas_call(
      body,
      grid=(_N // _RB,),
      in_specs=[pl.BlockSpec((_RB, K), lambda i: (i, 0)),
                pl.BlockSpec((_RB, K), lambda i: (i, 0)),
                pl.BlockSpec((_RB, 2 * _DH), lambda i: (i, 0)),
                pl.BlockSpec((1, K), lambda i: (0, 0)),
                pl.BlockSpec((K, Dout), lambda i: (0, 0))],
      out_specs=pl.BlockSpec((_RB, Dout), lambda i: (i, 0)),
      out_shape=jax.ShapeDtypeStruct((_N, Dout), jnp.float32),
  )(acc, hpp, g, b, W)


def _final(acc, hpp, g, b):
  def body(a_ref, h_ref, g_ref, b_ref, o_ref):
    g = g_ref[...][:, :h_ref.shape[1]]
    o_ref[...] = g * (a_ref[0] + a_ref[1] + h_ref[...]) + b_ref[...]

  D = hpp.shape[1]
  return pl.pallas_call(
      body,
      grid=(_N // _RB,),
      in_specs=[pl.BlockSpec((_NC, _RB, D), lambda i: (0, i, 0)),
                pl.BlockSpec((_RB, D), lambda i: (i, 0)),
                pl.BlockSpec((_RB, 2 * _DH), lambda i: (i, 0)),
                pl.BlockSpec((1, D), lambda i: (0, 0))],
      out_specs=pl.BlockSpec((_RB, D), lambda i: (i, 0)),
      out_shape=jax.ShapeDtypeStruct((_N, D), jnp.float32),
  )(acc, hpp, g, b)


def kernel(x, edge_index, W0, b0, W1, b1, W2, b2, W3, b3, W4, b4):
  pad_rw = jnp.zeros((_EPW - _E,), jnp.int32)
  pad_cw = jnp.full((_EPW - _E,), _NP - 1, jnp.int32)
  row_w = jnp.concatenate([edge_index[0], pad_rw])
  col_w = jnp.concatenate([edge_index[1], pad_cw])
  rowx2 = jnp.concatenate([2 * row_w, 2 * row_w + 1])
  pad_rn = jnp.zeros((_EPN - _E,), jnp.int32)
  pad_cn = jnp.full((_EPN - _E,), _NP - 1, jnp.int32)
  row_n = jnp.concatenate([edge_index[0], pad_rn])
  col_n = jnp.concatenate([edge_index[1], pad_cn])

  zeros64 = jnp.zeros((_RPT, _DH), jnp.float32)
  zeros16 = jnp.zeros((_RPT, 16), jnp.float32)
  ones16 = jnp.ones((_CHN, 16), jnp.float32)

  dacc = _deg_hist(col_n, ones16, zeros16)
  h0 = _mm_plain(x, W0)
  g, hpp = _g_scale(dacc, h0)

  W4p = jnp.pad(W4, ((0, 0), (0, 14)))
  b4p = jnp.pad(b4, (0, 14)).reshape(1, 16)

  bs = (b0.reshape(1, -1), b1.reshape(1, -1), b2.reshape(1, -1),
        b3.reshape(1, -1))
  Ws = (W1, W2, W3, W4p)
  for i in range(4):
    acc = _seg_sum_split(hpp.reshape(2 * _N, _DH), rowx2, col_w, zeros64)
    hpp = _mid(acc, hpp, g, bs[i], Ws[i])
  acc = _seg_sum_part(hpp, row_n, col_n, zeros16, 16)
  out16 = _final(acc, hpp, g, b4p)
  return out16[:, :2]


# speedup vs baseline: 1.3163x; 1.0713x over previous
import jax
import jax.numpy as jnp
from jax import lax
from jax.experimental import pallas as pl
from jax.experimental.pallas import tpu as pltpu
from jax.experimental.pallas import tpu_sc as plsc

_N = 10000
_E = 320000
_NC = 2
_NS = 16
_NW = _NC * _NS
_NP = 10240
_RPT = _NP // _NS
_DH = 64
_NB = 8
_DEPTH = 4
_RB = 2000

_CHW = 80
_EPW = _NS * _CHW * 250
_KJS = _EPW // _NS // _CHW

_CHN = 80
_EPN = _NW * _CHN * 125
_KJW = _EPN // _NW // _CHN

_mesh = plsc.VectorSubcoreMesh(core_axis_name="c", subcore_axis_name="s")
_sc_params = pltpu.CompilerParams(use_tc_tiling_on_sc=False)


def _ring_pipeline(tbl, row_v, col_v, acc_sh, msgs, gs, ss, kj, ch):
  def idx(ref, jj):
    return ref.at[pl.ds(jj * ch, ch)]

  def start_gather(jj, b):
    pltpu.async_copy(tbl.at[idx(row_v, jj)], msgs[b], gs[b])

  def wait_gather(jj, b):
    pltpu.make_async_copy(tbl.at[idx(row_v, jj)], msgs[b], gs[b]).wait()

  def start_scatter(jj, b):
    pltpu.async_copy(msgs[b], acc_sh.at[idx(col_v, jj)], ss[b], add=True)

  def drain_scatter(b):
    pltpu.make_async_copy(msgs[b], acc_sh.at[idx(col_v, 0)], ss[b]).wait()

  def step(jj, b, need_drain, need_start):
    wait_gather(jj, b)
    start_scatter(jj, b)
    if need_start:
      bd = (b + _DEPTH) % _NB
      if need_drain:
        drain_scatter(bd)
      start_gather(jj + _DEPTH, bd)

  for t in range(_DEPTH):
    start_gather(t, t)
  for jj in range(_NB):
    step(jj, jj % _NB, need_drain=(jj >= _NB - _DEPTH),
         need_start=(jj + _DEPTH < kj))

  main_hi = _NB + 8 * ((kj - _NB - _DEPTH) // 8)

  @pl.loop(_NB, main_hi, step=8)
  def _(jj0):
    for u in range(8):
      step(jj0 + u, u, need_drain=True, need_start=True)

  for jj in range(main_hi, kj):
    step(jj, jj % _NB, need_drain=True, need_start=(jj + _DEPTH < kj))
  for b in range(_NB):
    drain_scatter(b)


def _sc_scratch(kj, ch, d):
  return ([pltpu.VMEM((kj * ch,), jnp.int32),
           pltpu.VMEM((kj * ch,), jnp.int32)]
          + [pltpu.VMEM((ch, d), jnp.float32) for _ in range(_NB)]
          + [pltpu.SemaphoreType.DMA for _ in range(2 * _NB)])


def _seg_sum_split(hpp2, rowx2, col_f, zeros):
  @pl.kernel(
      out_type=jax.ShapeDtypeStruct((_NP, 2 * _DH), jnp.float32),
      mesh=_mesh,
      compiler_params=_sc_params,
      scratch_types=_sc_scratch(_KJS, _CHW, _DH)
      + [pltpu.VMEM_SHARED((_NP, _DH), jnp.float32)],
  )
  def k(hpp_hbm, row_hbm, col_hbm, zeros_hbm, out_hbm,
        row_v, col_v, *rest):
    msgs, gs, ss, acc_sh = (rest[:_NB], rest[_NB:2 * _NB],
                            rest[2 * _NB:3 * _NB], rest[3 * _NB])
    cid = lax.axis_index("c")
    sid = lax.axis_index("s")
    epw = _KJS * _CHW
    pltpu.sync_copy(zeros_hbm, acc_sh.at[pl.ds(sid * _RPT, _RPT)])
    pltpu.sync_copy(row_hbm.at[pl.ds(cid * _EPW + sid * epw, epw)], row_v)
    pltpu.sync_copy(col_hbm.at[pl.ds(sid * epw, epw)], col_v)
    plsc.subcore_barrier()
    _ring_pipeline(hpp_hbm, row_v, col_v, acc_sh, msgs, gs, ss,
                   _KJS, _CHW)
    plsc.subcore_barrier()
    pltpu.sync_copy(acc_sh.at[pl.ds(sid * _RPT, _RPT)],
                    out_hbm.at[pl.ds(sid * _RPT, _RPT),
                               pl.ds(cid * _DH, _DH)])

  return k(hpp2, rowx2, col_f, zeros)


def _seg_sum_part(hpp, row_f, col_f, zeros, D):
  @pl.kernel(
      out_type=jax.ShapeDtypeStruct((_NC, _NP, D), jnp.float32),
      mesh=_mesh,
      compiler_params=_sc_params,
      scratch_types=_sc_scratch(_KJW, _CHN, D)
      + [pltpu.VMEM_SHARED((_NP, D), jnp.float32)],
  )
  def k(hpp_hbm, row_hbm, col_hbm, zeros_hbm, out_hbm,
        row_v, col_v, *rest):
    msgs, gs, ss, acc_sh = (rest[:_NB], rest[_NB:2 * _NB],
                            rest[2 * _NB:3 * _NB], rest[3 * _NB])
    cid = lax.axis_index("c")
    sid = lax.axis_index("s")
    wid = sid * _NC + cid
    epw = _KJW * _CHN
    pltpu.sync_copy(zeros_hbm, acc_sh.at[pl.ds(sid * _RPT, _RPT)])
    pltpu.sync_copy(row_hbm.at[pl.ds(wid * epw, epw)], row_v)
    pltpu.sync_copy(col_hbm.at[pl.ds(wid * epw, epw)], col_v)
    plsc.subcore_barrier()
    _ring_pipeline(hpp_hbm, row_v, col_v, acc_sh, msgs, gs, ss,
                   _KJW, _CHN)
    plsc.subcore_barrier()
    pltpu.sync_copy(acc_sh.at[pl.ds(sid * _RPT, _RPT)],
                    out_hbm.at[cid, pl.ds(sid * _RPT, _RPT)])

  return k(hpp, row_f, col_f, zeros)


def _deg_hist(col_f, ones, zeros):
  @pl.kernel(
      out_type=jax.ShapeDtypeStruct((_NP, 2 * _DH), jnp.float32),
      mesh=_mesh,
      compiler_params=_sc_params,
      scratch_types=[
          pltpu.VMEM((_KJW * _CHN,), jnp.int32),
          pltpu.VMEM((_CHN, 16), jnp.float32),
          pltpu.SemaphoreType.DMA,
          pltpu.VMEM_SHARED((_NP, 16), jnp.float32),
      ],
  )
  def k(col_hbm, ones_hbm, zeros_hbm, out_hbm, col_v, ones_v, sem, acc_sh):
    cid = lax.axis_index("c")
    sid = lax.axis_index("s")
    wid = sid * _NC + cid
    epw = _KJW * _CHN
    pltpu.sync_copy(zeros_hbm, acc_sh.at[pl.ds(sid * _RPT, _RPT)])
    pltpu.sync_copy(col_hbm.at[pl.ds(wid * epw, epw)], col_v)
    pltpu.sync_copy(ones_hbm, ones_v)
    plsc.subcore_barrier()

    @pl.loop(0, _KJW)
    def _(j):
      pltpu.async_copy(ones_v, acc_sh.at[col_v.at[pl.ds(j * _CHN, _CHN)]],
                       sem, add=True)

    @pl.loop(0, _KJW)
    def _(j):
      pltpu.make_async_copy(ones_v, acc_sh.at[col_v.at[pl.ds(0, _CHN)]],
                            sem).wait()

    plsc.subcore_barrier()
    pltpu.sync_copy(acc_sh.at[pl.ds(sid * _RPT, _RPT)],
                    out_hbm.at[pl.ds(sid * _RPT, _RPT),
                               pl.ds(16 * cid, 16)])

  return k(col_f, ones, zeros)


def _mm_plain(x, W):
  def body(x_ref, w_ref, o_ref):
    o_ref[...] = lax.dot_general(x_ref[...], w_ref[...],
                                 (((1,), (0,)), ((), ())),
                                 preferred_element_type=jnp.float32,
                                 precision=lax.Precision.HIGHEST)

  K, Dout = W.shape
  return pl.pallas_call(
      body,
      grid=(_N // _RB,),
      in_specs=[pl.BlockSpec((_RB, K), lambda i: (i, 0)),
                pl.BlockSpec((K, Dout), lambda i: (0, 0))],
      out_specs=pl.BlockSpec((_RB, Dout), lambda i: (i, 0)),
      out_shape=jax.ShapeDtypeStruct((_N, Dout), jnp.float32),
  )(x, W)


def _g_scale(dacc, h0):
  def body(d_ref, h_ref, g_ref, o_ref):
    deg = d_ref[:, :1] + d_ref[:, 16:17] + 1.0
    g = jnp.broadcast_to(lax.rsqrt(deg), (_RB, 2 * _DH))
    g_ref[...] = g
    o_ref[...] = h_ref[...] * g

  return pl.pallas_call(
      body,
      grid=(_N // _RB,),
      in_specs=[pl.BlockSpec((_RB, 2 * _DH), lambda i: (i, 0)),
                pl.BlockSpec((_RB, 2 * _DH), lambda i: (i, 0))],
      out_specs=[pl.BlockSpec((_RB, 2 * _DH), lambda i: (i, 0)),
                 pl.BlockSpec((_RB, 2 * _DH), lambda i: (i, 0))],
      out_shape=[jax.ShapeDtypeStruct((_N, 2 * _DH), jnp.float32),
                 jax.ShapeDtypeStruct((_N, 2 * _DH), jnp.float32)],
  )(dacc, h0)


def _mid(acc, hpp, g, b, W):
  def body(a_ref, h_ref, g_ref, b_ref, w_ref, o_ref):
    g = g_ref[...]
    t = jnp.maximum(g * (a_ref[...] + h_ref[...]) + b_ref[...], 0.0)
    h = lax.dot_general(t, w_ref[...], (((1,), (0,)), ((), ())),
                        preferred_element_type=jnp.float32,
                        precision=lax.Precision.HIGHEST)
    o_ref[...] = h * g[:, :h.shape[1]]

  K, Dout = W.shape
  return pl.pallas_call(
      body,
      grid=(_N // _RB,),
      in_specs=[pl.BlockSpec((_RB, K), lambda i: (i, 0)),
                pl.BlockSpec((_RB, K), lambda i: (i, 0)),
                pl.BlockSpec((_RB, 2 * _DH), lambda i: (i, 0)),
                pl.BlockSpec((1, K), lambda i: (0, 0)),
                pl.BlockSpec((K, Dout), lambda i: (0, 0))],
      out_specs=pl.BlockSpec((_RB, Dout), lambda i: (i, 0)),
      out_shape=jax.ShapeDtypeStruct((_N, Dout), jnp.float32),
  )(acc, hpp, g, b, W)


def _final(acc, hpp, g, b):
  def body(a_ref, h_ref, g_ref, b_ref, o_ref):
    g = g_ref[...][:, :h_ref.shape[1]]
    o_ref[...] = g * (a_ref[0] + a_ref[1] + h_ref[...]) + b_ref[...]

  D = hpp.shape[1]
  return pl.pallas_call(
      body,
      grid=(_N // _RB,),
      in_specs=[pl.BlockSpec((_NC, _RB, D), lambda i: (0, i, 0)),
                pl.BlockSpec((_RB, D), lambda i: (i, 0)),
                pl.BlockSpec((_RB, 2 * _DH), lambda i: (i, 0)),
                pl.BlockSpec((1, D), lambda i: (0, 0))],
      out_specs=pl.BlockSpec((_RB, D), lambda i: (i, 0)),
      out_shape=jax.ShapeDtypeStruct((_N, D), jnp.float32),
  )(acc, hpp, g, b)


def kernel(x, edge_index, W0, b0, W1, b1, W2, b2, W3, b3, W4, b4):
  pad_rw = jnp.zeros((_EPW - _E,), jnp.int32)
  pad_cw = jnp.full((_EPW - _E,), _NP - 1, jnp.int32)
  row_w = jnp.concatenate([edge_index[0], pad_rw])
  col_w = jnp.concatenate([edge_index[1], pad_cw])
  rowx2 = jnp.concatenate([2 * row_w, 2 * row_w + 1])
  pad_rn = jnp.zeros((_EPN - _E,), jnp.int32)
  pad_cn = jnp.full((_EPN - _E,), _NP - 1, jnp.int32)
  row_n = jnp.concatenate([edge_index[0], pad_rn])
  col_n = jnp.concatenate([edge_index[1], pad_cn])

  zeros64 = jnp.zeros((_RPT, _DH), jnp.float32)
  zeros16 = jnp.zeros((_RPT, 16), jnp.float32)
  ones16 = jnp.ones((_CHN, 16), jnp.float32)

  dacc = _deg_hist(col_n, ones16, zeros16)
  h0 = _mm_plain(x, W0)
  g, hpp = _g_scale(dacc, h0)

  W4p = jnp.pad(W4, ((0, 0), (0, 14)))
  b4p = jnp.pad(b4, (0, 14)).reshape(1, 16)

  bs = (b0.reshape(1, -1), b1.reshape(1, -1), b2.reshape(1, -1),
        b3.reshape(1, -1))
  Ws = (W1, W2, W3, W4p)
  for i in range(4):
    acc = _seg_sum_split(hpp.reshape(2 * _N, _DH), rowx2, col_w, zeros64)
    hpp = _mid(acc, hpp, g, bs[i], Ws[i])
  acc = _seg_sum_part(hpp, row_n, col_n, zeros16, 16)
  out16 = _final(acc, hpp, g, b4p)
  return out16[:, :2]


# speedup vs baseline: 1.4742x; 1.1200x over previous
import jax
import jax.numpy as jnp
from jax import lax
from jax.experimental import pallas as pl
from jax.experimental.pallas import tpu as pltpu
from jax.experimental.pallas import tpu_sc as plsc

_N = 10000
_E = 320000
_NC = 2
_NS = 16
_NW = _NC * _NS
_NP = 10240
_RPT = _NP // _NS
_DH = 64
_NB = 8
_DEPTH = 6
_RB = 2000

_CHW = 80
_EPW = _NS * _CHW * 250
_KJS = _EPW // _NS // _CHW

_CHN = 80
_EPN = _NW * _CHN * 125
_KJW = _EPN // _NW // _CHN

_mesh = plsc.VectorSubcoreMesh(core_axis_name="c", subcore_axis_name="s")
_sc_params = pltpu.CompilerParams(use_tc_tiling_on_sc=False)


def _ring_pipeline(tbl, row_v, col_v, acc_sh, msgs, gs, ss, kj, ch):
  def idx(ref, jj):
    return ref.at[pl.ds(jj * ch, ch)]

  def start_gather(jj, b):
    pltpu.async_copy(tbl.at[idx(row_v, jj)], msgs[b], gs[b])

  def wait_gather(jj, b):
    pltpu.make_async_copy(tbl.at[idx(row_v, jj)], msgs[b], gs[b]).wait()

  def start_scatter(jj, b):
    pltpu.async_copy(msgs[b], acc_sh.at[idx(col_v, jj)], ss[b], add=True)

  def drain_scatter(b):
    pltpu.make_async_copy(msgs[b], acc_sh.at[idx(col_v, 0)], ss[b]).wait()

  def step(jj, b, need_drain, need_start):
    wait_gather(jj, b)
    start_scatter(jj, b)
    if need_start:
      bd = (b + _DEPTH) % _NB
      if need_drain:
        drain_scatter(bd)
      start_gather(jj + _DEPTH, bd)

  for t in range(_DEPTH):
    start_gather(t, t)
  for jj in range(_NB):
    step(jj, jj % _NB, need_drain=(jj >= _NB - _DEPTH),
         need_start=(jj + _DEPTH < kj))

  main_hi = _NB + 8 * ((kj - _NB - _DEPTH) // 8)

  @pl.loop(_NB, main_hi, step=8)
  def _(jj0):
    for u in range(8):
      step(jj0 + u, u, need_drain=True, need_start=True)

  for jj in range(main_hi, kj):
    step(jj, jj % _NB, need_drain=True, need_start=(jj + _DEPTH < kj))
  for b in range(_NB):
    drain_scatter(b)


def _sc_scratch(kj, ch, d):
  return ([pltpu.VMEM((kj * ch,), jnp.int32),
           pltpu.VMEM((kj * ch,), jnp.int32)]
          + [pltpu.VMEM((ch, d), jnp.float32) for _ in range(_NB)]
          + [pltpu.SemaphoreType.DMA for _ in range(2 * _NB)])


def _seg_sum_split(hpp2, rowx2, col_f, zeros):
  @pl.kernel(
      out_type=jax.ShapeDtypeStruct((_NP, 2 * _DH), jnp.float32),
      mesh=_mesh,
      compiler_params=_sc_params,
      scratch_types=_sc_scratch(_KJS, _CHW, _DH)
      + [pltpu.VMEM_SHARED((_NP, _DH), jnp.float32)],
  )
  def k(hpp_hbm, row_hbm, col_hbm, zeros_hbm, out_hbm,
        row_v, col_v, *rest):
    msgs, gs, ss, acc_sh = (rest[:_NB], rest[_NB:2 * _NB],
                            rest[2 * _NB:3 * _NB], rest[3 * _NB])
    cid = lax.axis_index("c")
    sid = lax.axis_index("s")
    epw = _KJS * _CHW
    pltpu.sync_copy(zeros_hbm, acc_sh.at[pl.ds(sid * _RPT, _RPT)])
    pltpu.sync_copy(row_hbm.at[pl.ds(cid * _EPW + sid * epw, epw)], row_v)
    pltpu.sync_copy(col_hbm.at[pl.ds(sid * epw, epw)], col_v)
    plsc.subcore_barrier()
    _ring_pipeline(hpp_hbm, row_v, col_v, acc_sh, msgs, gs, ss,
                   _KJS, _CHW)
    plsc.subcore_barrier()
    pltpu.sync_copy(acc_sh.at[pl.ds(sid * _RPT, _RPT)],
                    out_hbm.at[pl.ds(sid * _RPT, _RPT),
                               pl.ds(cid * _DH, _DH)])

  return k(hpp2, rowx2, col_f, zeros)


def _seg_sum_part(hpp, row_f, col_f, zeros, D):
  @pl.kernel(
      out_type=jax.ShapeDtypeStruct((_NC, _NP, D), jnp.float32),
      mesh=_mesh,
      compiler_params=_sc_params,
      scratch_types=_sc_scratch(_KJW, _CHN, D)
      + [pltpu.VMEM_SHARED((_NP, D), jnp.float32)],
  )
  def k(hpp_hbm, row_hbm, col_hbm, zeros_hbm, out_hbm,
        row_v, col_v, *rest):
    msgs, gs, ss, acc_sh = (rest[:_NB], rest[_NB:2 * _NB],
                            rest[2 * _NB:3 * _NB], rest[3 * _NB])
    cid = lax.axis_index("c")
    sid = lax.axis_index("s")
    wid = sid * _NC + cid
    epw = _KJW * _CHN
    pltpu.sync_copy(zeros_hbm, acc_sh.at[pl.ds(sid * _RPT, _RPT)])
    pltpu.sync_copy(row_hbm.at[pl.ds(wid * epw, epw)], row_v)
    pltpu.sync_copy(col_hbm.at[pl.ds(wid * epw, epw)], col_v)
    plsc.subcore_barrier()
    _ring_pipeline(hpp_hbm, row_v, col_v, acc_sh, msgs, gs, ss,
                   _KJW, _CHN)
    plsc.subcore_barrier()
    pltpu.sync_copy(acc_sh.at[pl.ds(sid * _RPT, _RPT)],
                    out_hbm.at[cid, pl.ds(sid * _RPT, _RPT)])

  return k(hpp, row_f, col_f, zeros)


def _deg_hist(col_f, ones, zeros):
  @pl.kernel(
      out_type=jax.ShapeDtypeStruct((_NP, 2 * _DH), jnp.float32),
      mesh=_mesh,
      compiler_params=_sc_params,
      scratch_types=[
          pltpu.VMEM((_KJW * _CHN,), jnp.int32),
          pltpu.VMEM((_CHN, 16), jnp.float32),
          pltpu.SemaphoreType.DMA,
          pltpu.VMEM_SHARED((_NP, 16), jnp.float32),
      ],
  )
  def k(col_hbm, ones_hbm, zeros_hbm, out_hbm, col_v, ones_v, sem, acc_sh):
    cid = lax.axis_index("c")
    sid = lax.axis_index("s")
    wid = sid * _NC + cid
    epw = _KJW * _CHN
    pltpu.sync_copy(zeros_hbm, acc_sh.at[pl.ds(sid * _RPT, _RPT)])
    pltpu.sync_copy(col_hbm.at[pl.ds(wid * epw, epw)], col_v)
    pltpu.sync_copy(ones_hbm, ones_v)
    plsc.subcore_barrier()

    @pl.loop(0, _KJW)
    def _(j):
      pltpu.async_copy(ones_v, acc_sh.at[col_v.at[pl.ds(j * _CHN, _CHN)]],
                       sem, add=True)

    @pl.loop(0, _KJW)
    def _(j):
      pltpu.make_async_copy(ones_v, acc_sh.at[col_v.at[pl.ds(0, _CHN)]],
                            sem).wait()

    plsc.subcore_barrier()
    pltpu.sync_copy(acc_sh.at[pl.ds(sid * _RPT, _RPT)],
                    out_hbm.at[pl.ds(sid * _RPT, _RPT),
                               pl.ds(16 * cid, 16)])

  return k(col_f, ones, zeros)


def _mm_plain(x, W):
  def body(x_ref, w_ref, o_ref):
    o_ref[...] = lax.dot_general(x_ref[...], w_ref[...],
                                 (((1,), (0,)), ((), ())),
                                 preferred_element_type=jnp.float32,
                                 precision=lax.Precision.HIGHEST)

  K, Dout = W.shape
  return pl.pallas_call(
      body,
      grid=(_N // _RB,),
      in_specs=[pl.BlockSpec((_RB, K), lambda i: (i, 0)),
                pl.BlockSpec((K, Dout), lambda i: (0, 0))],
      out_specs=pl.BlockSpec((_RB, Dout), lambda i: (i, 0)),
      out_shape=jax.ShapeDtypeStruct((_N, Dout), jnp.float32),
  )(x, W)


def _g_scale(dacc, h0):
  def body(d_ref, h_ref, g_ref, o_ref):
    deg = d_ref[:, :1] + d_ref[:, 16:17] + 1.0
    g = jnp.broadcast_to(lax.rsqrt(deg), (_RB, 2 * _DH))
    g_ref[...] = g
    o_ref[...] = h_ref[...] * g

  return pl.pallas_call(
      body,
      grid=(_N // _RB,),
      in_specs=[pl.BlockSpec((_RB, 2 * _DH), lambda i: (i, 0)),
                pl.BlockSpec((_RB, 2 * _DH), lambda i: (i, 0))],
      out_specs=[pl.BlockSpec((_RB, 2 * _DH), lambda i: (i, 0)),
                 pl.BlockSpec((_RB, 2 * _DH), lambda i: (i, 0))],
      out_shape=[jax.ShapeDtypeStruct((_N, 2 * _DH), jnp.float32),
                 jax.ShapeDtypeStruct((_N, 2 * _DH), jnp.float32)],
  )(dacc, h0)


def _mid(acc, hpp, g, b, W):
  def body(a_ref, h_ref, g_ref, b_ref, w_ref, o_ref):
    g = g_ref[...]
    t = jnp.maximum(g * (a_ref[...] + h_ref[...]) + b_ref[...], 0.0)
    h = lax.dot_general(t, w_ref[...], (((1,), (0,)), ((), ())),
                        preferred_element_type=jnp.float32,
                        precision=lax.Precision.HIGHEST)
    o_ref[...] = h * g[:, :h.shape[1]]

  K, Dout = W.shape
  return pl.pallas_call(
      body,
      grid=(_N // _RB,),
      in_specs=[pl.BlockSpec((_RB, K), lambda i: (i, 0)),
                pl.BlockSpec((_RB, K), lambda i: (i, 0)),
                pl.BlockSpec((_RB, 2 * _DH), lambda i: (i, 0)),
                pl.BlockSpec((1, K), lambda i: (0, 0)),
                pl.BlockSpec((K, Dout), lambda i: (0, 0))],
      out_specs=pl.BlockSpec((_RB, Dout), lambda i: (i, 0)),
      out_shape=jax.ShapeDtypeStruct((_N, Dout), jnp.float32),
  )(acc, hpp, g, b, W)


def _final(acc, hpp, g, b):
  def body(a_ref, h_ref, g_ref, b_ref, o_ref):
    g = g_ref[...][:, :h_ref.shape[1]]
    o_ref[...] = g * (a_ref[0] + a_ref[1] + h_ref[...]) + b_ref[...]

  D = hpp.shape[1]
  return pl.pallas_call(
      body,
      grid=(_N // _RB,),
      in_specs=[pl.BlockSpec((_NC, _RB, D), lambda i: (0, i, 0)),
                pl.BlockSpec((_RB, D), lambda i: (i, 0)),
                pl.BlockSpec((_RB, 2 * _DH), lambda i: (i, 0)),
                pl.BlockSpec((1, D), lambda i: (0, 0))],
      out_specs=pl.BlockSpec((_RB, D), lambda i: (i, 0)),
      out_shape=jax.ShapeDtypeStruct((_N, D), jnp.float32),
  )(acc, hpp, g, b)


def kernel(x, edge_index, W0, b0, W1, b1, W2, b2, W3, b3, W4, b4):
  pad_rw = jnp.zeros((_EPW - _E,), jnp.int32)
  pad_cw = jnp.full((_EPW - _E,), _NP - 1, jnp.int32)
  row_w = jnp.concatenate([edge_index[0], pad_rw])
  col_w = jnp.concatenate([edge_index[1], pad_cw])
  rowx2 = jnp.concatenate([2 * row_w, 2 * row_w + 1])
  pad_rn = jnp.zeros((_EPN - _E,), jnp.int32)
  pad_cn = jnp.full((_EPN - _E,), _NP - 1, jnp.int32)
  row_n = jnp.concatenate([edge_index[0], pad_rn])
  col_n = jnp.concatenate([edge_index[1], pad_cn])

  zeros64 = jnp.zeros((_RPT, _DH), jnp.float32)
  zeros16 = jnp.zeros((_RPT, 16), jnp.float32)
  ones16 = jnp.ones((_CHN, 16), jnp.float32)

  dacc = _deg_hist(col_n, ones16, zeros16)
  h0 = _mm_plain(x, W0)
  g, hpp = _g_scale(dacc, h0)

  W4p = jnp.pad(W4, ((0, 0), (0, 14)))
  b4p = jnp.pad(b4, (0, 14)).reshape(1, 16)

  bs = (b0.reshape(1, -1), b1.reshape(1, -1), b2.reshape(1, -1),
        b3.reshape(1, -1))
  Ws = (W1, W2, W3, W4p)
  for i in range(4):
    acc = _seg_sum_split(hpp.reshape(2 * _N, _DH), rowx2, col_w, zeros64)
    hpp = _mid(acc, hpp, g, bs[i], Ws[i])
  acc = _seg_sum_part(hpp, row_n, col_n, zeros16, 16)
  out16 = _final(acc, hpp, g, b4p)
  return out16[:, :2]


# speedup vs baseline: 1.4763x; 1.0015x over previous
import jax
import jax.numpy as jnp
from jax import lax
from jax.experimental import pallas as pl
from jax.experimental.pallas import tpu as pltpu
from jax.experimental.pallas import tpu_sc as plsc

_N = 10000
_E = 320000
_NC = 2
_NS = 16
_NW = _NC * _NS
_NP = 10240
_RPT = _NP // _NS
_DH = 64
_NB = 8
_DEPTH = 7
_RB = 2000

_CHW = 80
_EPW = _NS * _CHW * 250
_KJS = _EPW // _NS // _CHW

_CHN = 80
_EPN = _NW * _CHN * 125
_KJW = _EPN // _NW // _CHN

_mesh = plsc.VectorSubcoreMesh(core_axis_name="c", subcore_axis_name="s")
_sc_params = pltpu.CompilerParams(use_tc_tiling_on_sc=False)


def _ring_pipeline(tbl, row_v, col_v, acc_sh, msgs, gs, ss, kj, ch):
  def idx(ref, jj):
    return ref.at[pl.ds(jj * ch, ch)]

  def start_gather(jj, b):
    pltpu.async_copy(tbl.at[idx(row_v, jj)], msgs[b], gs[b])

  def wait_gather(jj, b):
    pltpu.make_async_copy(tbl.at[idx(row_v, jj)], msgs[b], gs[b]).wait()

  def start_scatter(jj, b):
    pltpu.async_copy(msgs[b], acc_sh.at[idx(col_v, jj)], ss[b], add=True)

  def drain_scatter(b):
    pltpu.make_async_copy(msgs[b], acc_sh.at[idx(col_v, 0)], ss[b]).wait()

  def step(jj, b, need_drain, need_start):
    wait_gather(jj, b)
    start_scatter(jj, b)
    if need_start:
      bd = (b + _DEPTH) % _NB
      if need_drain:
        drain_scatter(bd)
      start_gather(jj + _DEPTH, bd)

  for t in range(_DEPTH):
    start_gather(t, t)
  for jj in range(_NB):
    step(jj, jj % _NB, need_drain=(jj >= _NB - _DEPTH),
         need_start=(jj + _DEPTH < kj))

  main_hi = _NB + 8 * ((kj - _NB - _DEPTH) // 8)

  @pl.loop(_NB, main_hi, step=8)
  def _(jj0):
    for u in range(8):
      step(jj0 + u, u, need_drain=True, need_start=True)

  for jj in range(main_hi, kj):
    step(jj, jj % _NB, need_drain=True, need_start=(jj + _DEPTH < kj))
  for b in range(_NB):
    drain_scatter(b)


def _sc_scratch(kj, ch, d):
  return ([pltpu.VMEM((kj * ch,), jnp.int32),
           pltpu.VMEM((kj * ch,), jnp.int32)]
          + [pltpu.VMEM((ch, d), jnp.float32) for _ in range(_NB)]
          + [pltpu.SemaphoreType.DMA for _ in range(2 * _NB)])


def _seg_sum_split(hpp2, rowx2, col_f, zeros):
  @pl.kernel(
      out_type=jax.ShapeDtypeStruct((_NP, 2 * _DH), jnp.float32),
      mesh=_mesh,
      compiler_params=_sc_params,
      scratch_types=_sc_scratch(_KJS, _CHW, _DH)
      + [pltpu.VMEM_SHARED((_NP, _DH), jnp.float32)],
  )
  def k(hpp_hbm, row_hbm, col_hbm, zeros_hbm, out_hbm,
        row_v, col_v, *rest):
    msgs, gs, ss, acc_sh = (rest[:_NB], rest[_NB:2 * _NB],
                            rest[2 * _NB:3 * _NB], rest[3 * _NB])
    cid = lax.axis_index("c")
    sid = lax.axis_index("s")
    epw = _KJS * _CHW
    pltpu.sync_copy(zeros_hbm, acc_sh.at[pl.ds(sid * _RPT, _RPT)])
    pltpu.sync_copy(row_hbm.at[pl.ds(cid * _EPW + sid * epw, epw)], row_v)
    pltpu.sync_copy(col_hbm.at[pl.ds(sid * epw, epw)], col_v)
    plsc.subcore_barrier()
    _ring_pipeline(hpp_hbm, row_v, col_v, acc_sh, msgs, gs, ss,
                   _KJS, _CHW)
    plsc.subcore_barrier()
    pltpu.sync_copy(acc_sh.at[pl.ds(sid * _RPT, _RPT)],
                    out_hbm.at[pl.ds(sid * _RPT, _RPT),
                               pl.ds(cid * _DH, _DH)])

  return k(hpp2, rowx2, col_f, zeros)


def _seg_sum_part(hpp, row_f, col_f, zeros, D):
  @pl.kernel(
      out_type=jax.ShapeDtypeStruct((_NC, _NP, D), jnp.float32),
      mesh=_mesh,
      compiler_params=_sc_params,
      scratch_types=_sc_scratch(_KJW, _CHN, D)
      + [pltpu.VMEM_SHARED((_NP, D), jnp.float32)],
  )
  def k(hpp_hbm, row_hbm, col_hbm, zeros_hbm, out_hbm,
        row_v, col_v, *rest):
    msgs, gs, ss, acc_sh = (rest[:_NB], rest[_NB:2 * _NB],
                            rest[2 * _NB:3 * _NB], rest[3 * _NB])
    cid = lax.axis_index("c")
    sid = lax.axis_index("s")
    wid = sid * _NC + cid
    epw = _KJW * _CHN
    pltpu.sync_copy(zeros_hbm, acc_sh.at[pl.ds(sid * _RPT, _RPT)])
    pltpu.sync_copy(row_hbm.at[pl.ds(wid * epw, epw)], row_v)
    pltpu.sync_copy(col_hbm.at[pl.ds(wid * epw, epw)], col_v)
    plsc.subcore_barrier()
    _ring_pipeline(hpp_hbm, row_v, col_v, acc_sh, msgs, gs, ss,
                   _KJW, _CHN)
    plsc.subcore_barrier()
    pltpu.sync_copy(acc_sh.at[pl.ds(sid * _RPT, _RPT)],
                    out_hbm.at[cid, pl.ds(sid * _RPT, _RPT)])

  return k(hpp, row_f, col_f, zeros)


def _deg_hist(col_f, ones, zeros):
  @pl.kernel(
      out_type=jax.ShapeDtypeStruct((_NP, 2 * _DH), jnp.float32),
      mesh=_mesh,
      compiler_params=_sc_params,
      scratch_types=[
          pltpu.VMEM((_KJW * _CHN,), jnp.int32),
          pltpu.VMEM((_CHN, 16), jnp.float32),
          pltpu.SemaphoreType.DMA,
          pltpu.VMEM_SHARED((_NP, 16), jnp.float32),
      ],
  )
  def k(col_hbm, ones_hbm, zeros_hbm, out_hbm, col_v, ones_v, sem, acc_sh):
    cid = lax.axis_index("c")
    sid = lax.axis_index("s")
    wid = sid * _NC + cid
    epw = _KJW * _CHN
    pltpu.sync_copy(zeros_hbm, acc_sh.at[pl.ds(sid * _RPT, _RPT)])
    pltpu.sync_copy(col_hbm.at[pl.ds(wid * epw, epw)], col_v)
    pltpu.sync_copy(ones_hbm, ones_v)
    plsc.subcore_barrier()

    @pl.loop(0, _KJW)
    def _(j):
      pltpu.async_copy(ones_v, acc_sh.at[col_v.at[pl.ds(j * _CHN, _CHN)]],
                       sem, add=True)

    @pl.loop(0, _KJW)
    def _(j):
      pltpu.make_async_copy(ones_v, acc_sh.at[col_v.at[pl.ds(0, _CHN)]],
                            sem).wait()

    plsc.subcore_barrier()
    pltpu.sync_copy(acc_sh.at[pl.ds(sid * _RPT, _RPT)],
                    out_hbm.at[pl.ds(sid * _RPT, _RPT),
                               pl.ds(16 * cid, 16)])

  return k(col_f, ones, zeros)


def _mm_plain(x, W):
  def body(x_ref, w_ref, o_ref):
    o_ref[...] = lax.dot_general(x_ref[...], w_ref[...],
                                 (((1,), (0,)), ((), ())),
                                 preferred_element_type=jnp.float32,
                                 precision=lax.Precision.HIGHEST)

  K, Dout = W.shape
  return pl.pallas_call(
      body,
      grid=(_N // _RB,),
      in_specs=[pl.BlockSpec((_RB, K), lambda i: (i, 0)),
                pl.BlockSpec((K, Dout), lambda i: (0, 0))],
      out_specs=pl.BlockSpec((_RB, Dout), lambda i: (i, 0)),
      out_shape=jax.ShapeDtypeStruct((_N, Dout), jnp.float32),
  )(x, W)


def _g_scale(dacc, h0):
  def body(d_ref, h_ref, g_ref, o_ref):
    deg = d_ref[:, :1] + d_ref[:, 16:17] + 1.0
    g = jnp.broadcast_to(lax.rsqrt(deg), (_RB, 2 * _DH))
    g_ref[...] = g
    o_ref[...] = h_ref[...] * g

  return pl.pallas_call(
      body,
      grid=(_N // _RB,),
      in_specs=[pl.BlockSpec((_RB, 2 * _DH), lambda i: (i, 0)),
                pl.BlockSpec((_RB, 2 * _DH), lambda i: (i, 0))],
      out_specs=[pl.BlockSpec((_RB, 2 * _DH), lambda i: (i, 0)),
                 pl.BlockSpec((_RB, 2 * _DH), lambda i: (i, 0))],
      out_shape=[jax.ShapeDtypeStruct((_N, 2 * _DH), jnp.float32),
                 jax.ShapeDtypeStruct((_N, 2 * _DH), jnp.float32)],
  )(dacc, h0)


def _mid(acc, hpp, g, b, W):
  def body(a_ref, h_ref, g_ref, b_ref, w_ref, o_ref):
    g = g_ref[...]
    t = jnp.maximum(g * (a_ref[...] + h_ref[...]) + b_ref[...], 0.0)
    h = lax.dot_general(t, w_ref[...], (((1,), (0,)), ((), ())),
                        preferred_element_type=jnp.float32,
                        precision=lax.Precision.HIGHEST)
    o_ref[...] = h * g[:, :h.shape[1]]

  K, Dout = W.shape
  return pl.pallas_call(
      body,
      grid=(_N // _RB,),
      in_specs=[pl.BlockSpec((_RB, K), lambda i: (i, 0)),
                pl.BlockSpec((_RB, K), lambda i: (i, 0)),
                pl.BlockSpec((_RB, 2 * _DH), lambda i: (i, 0)),
                pl.BlockSpec((1, K), lambda i: (0, 0)),
                pl.BlockSpec((K, Dout), lambda i: (0, 0))],
      out_specs=pl.BlockSpec((_RB, Dout), lambda i: (i, 0)),
      out_shape=jax.ShapeDtypeStruct((_N, Dout), jnp.float32),
  )(acc, hpp, g, b, W)


def _final(acc, hpp, g, b):
  def body(a_ref, h_ref, g_ref, b_ref, o_ref):
    g = g_ref[...][:, :h_ref.shape[1]]
    o_ref[...] = g * (a_ref[0] + a_ref[1] + h_ref[...]) + b_ref[...]

  D = hpp.shape[1]
  return pl.pallas_call(
      body,
      grid=(_N // _RB,),
      in_specs=[pl.BlockSpec((_NC, _RB, D), lambda i: (0, i, 0)),
                pl.BlockSpec((_RB, D), lambda i: (i, 0)),
                pl.BlockSpec((_RB, 2 * _DH), lambda i: (i, 0)),
                pl.BlockSpec((1, D), lambda i: (0, 0))],
      out_specs=pl.BlockSpec((_RB, D), lambda i: (i, 0)),
      out_shape=jax.ShapeDtypeStruct((_N, D), jnp.float32),
  )(acc, hpp, g, b)


def kernel(x, edge_index, W0, b0, W1, b1, W2, b2, W3, b3, W4, b4):
  pad_rw = jnp.zeros((_EPW - _E,), jnp.int32)
  pad_cw = jnp.full((_EPW - _E,), _NP - 1, jnp.int32)
  row_w = jnp.concatenate([edge_index[0], pad_rw])
  col_w = jnp.concatenate([edge_index[1], pad_cw])
  rowx2 = jnp.concatenate([2 * row_w, 2 * row_w + 1])
  pad_rn = jnp.zeros((_EPN - _E,), jnp.int32)
  pad_cn = jnp.full((_EPN - _E,), _NP - 1, jnp.int32)
  row_n = jnp.concatenate([edge_index[0], pad_rn])
  col_n = jnp.concatenate([edge_index[1], pad_cn])

  zeros64 = jnp.zeros((_RPT, _DH), jnp.float32)
  zeros16 = jnp.zeros((_RPT, 16), jnp.float32)
  ones16 = jnp.ones((_CHN, 16), jnp.float32)

  dacc = _deg_hist(col_n, ones16, zeros16)
  h0 = _mm_plain(x, W0)
  g, hpp = _g_scale(dacc, h0)

  W4p = jnp.pad(W4, ((0, 0), (0, 14)))
  b4p = jnp.pad(b4, (0, 14)).reshape(1, 16)

  bs = (b0.reshape(1, -1), b1.reshape(1, -1), b2.reshape(1, -1),
        b3.reshape(1, -1))
  Ws = (W1, W2, W3, W4p)
  for i in range(4):
    acc = _seg_sum_split(hpp.reshape(2 * _N, _DH), rowx2, col_w, zeros64)
    hpp = _mid(acc, hpp, g, bs[i], Ws[i])
  acc = _seg_sum_part(hpp, row_n, col_n, zeros16, 16)
  out16 = _final(acc, hpp, g, b4p)
  return out16[:, :2]
